# baseline (speedup 1.0000x reference)
import functools

import jax
import jax.numpy as jnp
from jax import lax
from jax.experimental import pallas as pl
from jax.experimental.pallas import tpu as pltpu
from jax.experimental.pallas import tpu_sc as plsc

BATCH = 4096
SEQ = 200
EMBED_DIM = 64
LANES = 128
VOCAB = 1000000

NUM_CORES = 2
NUM_SUBCORES = 16
NUM_WORKERS = NUM_CORES * NUM_SUBCORES

N_BC = BATCH // LANES
NBUF = 5
N_GROUPS = SEQ // NBUF
KG = LANES // 16
DR = EMBED_DIM // 8


def _make_kernel():
    mesh = plsc.VectorSubcoreMesh(core_axis_name="c", subcore_axis_name="s")

    @functools.partial(
        pl.kernel,
        mesh=mesh,
        out_type=jax.ShapeDtypeStruct((SEQ, DR, N_BC, 8, LANES), jnp.float32),
        scratch_types=[
            pltpu.VMEM((SEQ, LANES), jnp.int32),
            pltpu.VMEM((NBUF, LANES, EMBED_DIM), jnp.float32),
            pltpu.VMEM((NBUF, EMBED_DIM, LANES + 1), jnp.float32),
        ]
        + [pltpu.SemaphoreType.DMA] * (2 * NBUF),
        compiler_params=pltpu.CompilerParams(
            use_tc_tiling_on_sc=False, needs_layout_passes=False
        ),
    )
    def emb(idx_hbm, table_hbm, out_hbm, idx_all, rows_v, tv_v, *sems):
        gsem = sems[:NBUF]
        ssem = sems[NBUF:]
        wid = lax.axis_index("s") * NUM_CORES + lax.axis_index("c")

        pltpu.sync_copy(idx_hbm.at[wid], idx_all)

        iota16 = lax.iota(jnp.int32, 16)
        drow = [iota16 + (dg * 16) for dg in range(EMBED_DIM // 16)]
        zerov = iota16 * 0

        def gather(s, b):
            return pltpu.async_copy(
                table_hbm.at[idx_all.at[s]], rows_v.at[b], gsem[b]
            )

        def gather_wait(b):
            pltpu.make_async_copy(
                table_hbm.at[idx_all.at[0]], rows_v.at[b], gsem[b]
            ).wait()

        def transpose(b):
            @plsc.parallel_loop(0, LANES, 1, unroll=16)
            def kloop(k):
                kv = zerov + k
                for dg in range(EMBED_DIM // 16):
                    vals = rows_v[b, k, pl.ds(dg * 16, 16)]
                    plsc.store_scatter(tv_v.at[b], [drow[dg], kv], vals)

        def store_out(s, b):
            for dr in range(DR):
                pltpu.async_copy(
                    tv_v.at[b, pl.ds(dr * 8, 8), pl.ds(0, LANES)],
                    out_hbm.at[s, dr, wid],
                    ssem[b],
                )

        def store_wait(b):
            for dr in range(DR):
                pltpu.make_async_copy(
                    tv_v.at[b, pl.ds(dr * 8, 8), pl.ds(0, LANES)],
                    out_hbm.at[0, dr, 0],
                    ssem[b],
                ).wait()

        for b in range(NBUF):
            gather(b, b)

        def body(g, carry):
            for b in range(NBUF):
                gather_wait(b)
                transpose(b)
                store_out(g * NBUF + b, b)
            for b in range(NBUF):
                store_wait(b)
                gather((g + 1) * NBUF + b, b)
            return carry

        lax.fori_loop(0, N_GROUPS - 1, body, 0)

        for b in range(NBUF):
            gather_wait(b)
            transpose(b)
            store_out((N_GROUPS - 1) * NBUF + b, b)
        for b in range(NBUF):
            store_wait(b)

    return emb


_emb = _make_kernel()


def kernel(input, table):
    idx3 = input.T.reshape(SEQ, N_BC, LANES).transpose(1, 0, 2)
    out5 = _emb(idx3, table)
    return out5.transpose(2, 4, 0, 1, 3).reshape(BATCH, SEQ, EMBED_DIM)


# speedup vs baseline: 1.0594x; 1.0594x over previous
import functools

import jax
import jax.numpy as jnp
from jax import lax
from jax.experimental import pallas as pl
from jax.experimental.pallas import tpu as pltpu
from jax.experimental.pallas import tpu_sc as plsc

BATCH = 4096
SEQ = 200
EMBED_DIM = 64
LANES = 128
VOCAB = 1000000

NUM_CORES = 2
NUM_SUBCORES = 16
NUM_WORKERS = NUM_CORES * NUM_SUBCORES

N_BC = BATCH // LANES
NBUF = 5
N_GROUPS = SEQ // NBUF
KG = LANES // 16
DR = EMBED_DIM // 8


def _make_kernel():
    mesh = plsc.VectorSubcoreMesh(core_axis_name="c", subcore_axis_name="s")

    @functools.partial(
        pl.kernel,
        mesh=mesh,
        out_type=jax.ShapeDtypeStruct((SEQ, DR, N_BC, 8, LANES), jnp.float32),
        scratch_types=[
            pltpu.VMEM((SEQ, LANES), jnp.int32),
            pltpu.VMEM((NBUF, LANES, EMBED_DIM), jnp.float32),
            pltpu.VMEM((NBUF, EMBED_DIM, LANES + 1), jnp.float32),
        ]
        + [pltpu.SemaphoreType.DMA] * (2 * NBUF),
        compiler_params=pltpu.CompilerParams(
            use_tc_tiling_on_sc=False, needs_layout_passes=False
        ),
    )
    def emb(idx_hbm, table_hbm, out_hbm, idx_all, rows_v, tv_v, *sems):
        gsem = sems[:NBUF]
        ssem = sems[NBUF:]
        wid = lax.axis_index("s") * NUM_CORES + lax.axis_index("c")

        pltpu.sync_copy(idx_hbm.at[wid], idx_all)

        iota16 = lax.iota(jnp.int32, 16)
        drow = [iota16 + (dg * 16) for dg in range(EMBED_DIM // 16)]
        zerov = iota16 * 0

        def gather(s, b):
            return pltpu.async_copy(
                table_hbm.at[idx_all.at[s]], rows_v.at[b], gsem[b]
            )

        def gather_wait(b):
            pltpu.make_async_copy(
                table_hbm.at[idx_all.at[0]], rows_v.at[b], gsem[b]
            ).wait()

        def transpose(b):
            @plsc.parallel_loop(0, LANES, 1, unroll=8)
            def kloop(k):
                kv = zerov + k
                for dg in range(EMBED_DIM // 16):
                    vals = rows_v[b, k, pl.ds(dg * 16, 16)]
                    plsc.store_scatter(tv_v.at[b], [drow[dg], kv], vals)

        def store_out(s, b):
            for dr in range(DR):
                pltpu.async_copy(
                    tv_v.at[b, pl.ds(dr * 8, 8), pl.ds(0, LANES)],
                    out_hbm.at[s, dr, wid],
                    ssem[b],
                )

        def store_wait(b):
            for dr in range(DR):
                pltpu.make_async_copy(
                    tv_v.at[b, pl.ds(dr * 8, 8), pl.ds(0, LANES)],
                    out_hbm.at[0, dr, 0],
                    ssem[b],
                ).wait()

        for b in range(NBUF):
            gather(b, b)

        def body(g, carry):
            for b in range(NBUF):
                gather_wait(b)
                transpose(b)
                store_out(g * NBUF + b, b)
            for b in range(NBUF):
                store_wait(b)
                gather((g + 1) * NBUF + b, b)
            return carry

        lax.fori_loop(0, N_GROUPS - 1, body, 0)

        for b in range(NBUF):
            gather_wait(b)
            transpose(b)
            store_out((N_GROUPS - 1) * NBUF + b, b)
        for b in range(NBUF):
            store_wait(b)

    return emb


_emb = _make_kernel()


def kernel(input, table):
    idx3 = input.T.reshape(SEQ, N_BC, LANES).transpose(1, 0, 2)
    out5 = _emb(idx3, table)
    return out5.transpose(2, 4, 0, 1, 3).reshape(BATCH, SEQ, EMBED_DIM)
